# Initial kernel scaffold; baseline (speedup 1.0000x reference)
import jax
import jax.numpy as jnp
from jax.experimental import pallas as pl


def kernel(x, cw0, cb0, cw1, cb1, cw2, cb2, cw3, cb3, cw4, cb4, w1, b1, w2, b2):
    raise NotImplementedError("write your pallas kernel here")


# speedup vs baseline: 9.9372x; 9.9372x over previous
import functools

import jax
import jax.numpy as jnp
from jax.experimental import pallas as pl
from jax.experimental.pallas import tpu as pltpu


def _s2d_conv_pool_kernel(z_ref, w_ref, b_ref, o_ref, col_ref):
    hp, wp, cout = o_ref.shape[1], o_ref.shape[2], o_ref.shape[3]
    c4 = z_ref.shape[3]
    mp = hp * wp

    for t, (p, q) in enumerate(((0, 0), (0, 1), (1, 0), (1, 1))):
        col_ref[:, t * c4:(t + 1) * c4] = (
            z_ref[0, p:p + hp, q:q + wp, :].reshape(mp, c4))

    acc = jnp.dot(col_ref[...], w_ref[...], preferred_element_type=jnp.float32)

    m = jnp.maximum(jnp.maximum(acc[:, :cout], acc[:, cout:2 * cout]),
                    jnp.maximum(acc[:, 2 * cout:3 * cout], acc[:, 3 * cout:]))
    m = jnp.maximum(m + b_ref[...], 0.0)
    o_ref[0] = m.reshape(hp, wp, cout).astype(o_ref.dtype)


def _s2d_conv_pool(z, w2, bias, hp, wp, cout, vmem_mb):
    B = z.shape[0]
    k16 = w2.shape[0]
    return pl.pallas_call(
        _s2d_conv_pool_kernel,
        out_shape=jax.ShapeDtypeStruct((B, hp, wp, cout), jnp.bfloat16),
        grid=(B,),
        in_specs=[
            pl.BlockSpec((1, z.shape[1], z.shape[2], z.shape[3]),
                         lambda b: (b, 0, 0, 0)),
            pl.BlockSpec((k16, 4 * cout), lambda b: (0, 0)),
            pl.BlockSpec((1, cout), lambda b: (0, 0)),
        ],
        out_specs=pl.BlockSpec((1, hp, wp, cout), lambda b: (b, 0, 0, 0)),
        scratch_shapes=[pltpu.VMEM((hp * wp, k16), jnp.bfloat16)],
        compiler_params=pltpu.CompilerParams(
            dimension_semantics=("parallel",),
            vmem_limit_bytes=vmem_mb * 1024 * 1024,
        ),
    )(z, w2, bias.reshape(1, cout).astype(jnp.float32))


def _fold_weights(w):
    cin, cout = w.shape[2], w.shape[3]
    zero = jnp.zeros((cin, cout), w.dtype)
    taps = []
    for p in (0, 1):
        for q in (0, 1):
            rows = []
            for a in (0, 1):
                for b in (0, 1):
                    cols = []
                    for dh in (0, 1):
                        for dw in (0, 1):
                            kh = 2 * p + a - dh
                            kw = 2 * q + b - dw
                            ok = 0 <= kh <= 2 and 0 <= kw <= 2
                            cols.append(w[kh, kw] if ok else zero)
                    rows.append(jnp.concatenate(cols, axis=1))
            taps.append(jnp.concatenate(rows, axis=0))
    return jnp.concatenate(taps, axis=0).astype(jnp.bfloat16)


def _s2d_nhwc(y):
    B, H, W, C = y.shape
    yp = jnp.pad(y, ((0, 0), (1, 1), (1, 1), (0, 0)))
    z = yp.reshape(B, (H + 2) // 2, 2, (W + 2) // 2, 2, C)
    return jnp.transpose(z, (0, 1, 3, 2, 4, 5)).reshape(
        B, (H + 2) // 2, (W + 2) // 2, 4 * C)


def _mlp_kernel(x_ref, w1_ref, b1_ref, w2_ref, b2_ref, o_ref):
    h = jnp.dot(x_ref[...], w1_ref[...], preferred_element_type=jnp.float32)
    h = jnp.maximum(h + b1_ref[...], 0.0)
    o_ref[...] = jnp.dot(h, w2_ref[...],
                         preferred_element_type=jnp.float32) + b2_ref[...]


def _mlp(x, w1p, b1, w2, b2):
    B, K = x.shape
    n1, n2 = w1p.shape[1], w2.shape[1]
    return pl.pallas_call(
        _mlp_kernel,
        out_shape=jax.ShapeDtypeStruct((B, n2), jnp.float32),
        grid=(1,),
        in_specs=[
            pl.BlockSpec((B, K), lambda i: (0, 0)),
            pl.BlockSpec((K, n1), lambda i: (0, 0)),
            pl.BlockSpec((1, n1), lambda i: (0, 0)),
            pl.BlockSpec((n1, n2), lambda i: (0, 0)),
            pl.BlockSpec((1, n2), lambda i: (0, 0)),
        ],
        out_specs=pl.BlockSpec((B, n2), lambda i: (0, 0)),
        compiler_params=pltpu.CompilerParams(
            dimension_semantics=("arbitrary",),
        ),
    )(x, w1p, b1.reshape(1, n1).astype(jnp.float32),
      w2.astype(jnp.float32), b2.reshape(1, n2).astype(jnp.float32))


@functools.partial(jax.jit, static_argnums=())
def kernel(x, cw0, cb0, cw1, cb1, cw2, cb2, cw3, cb3, cw4, cb4, w1, b1, w2, b2):
    B, _, H, W = x.shape

    xb = x.astype(jnp.bfloat16)
    xp = jnp.pad(xb, ((0, 0), (0, 0), (1, 1), (1, 1)))
    z = xp.reshape(B, 3, (H + 2) // 2, 2, (W + 2) // 2, 2)
    z = jnp.transpose(z, (0, 2, 4, 3, 5, 1))
    z = z.reshape(B, (H + 2) // 2, (W + 2) // 2, 12)

    convs = [(cw0, cb0), (cw1, cb1), (cw2, cb2), (cw3, cb3), (cw4, cb4)]
    hp, wp = H // 2, W // 2
    vmem = (48, 32, 32, 32, 32)
    for i, (cw, cb) in enumerate(convs):
        w2d = _fold_weights(cw)
        y = _s2d_conv_pool(z, w2d, cb, hp, wp, cw.shape[3], vmem[i])
        if i < len(convs) - 1:
            z = _s2d_nhwc(y)
            hp, wp = hp // 2, wp // 2

    feat = y.reshape(B, -1)
    cout = cw4.shape[3]
    hw = y.shape[1] * y.shape[2]
    w1p = (w1.reshape(cout, y.shape[1], y.shape[2], w1.shape[1])
           .transpose(1, 2, 0, 3).reshape(hw * cout, w1.shape[1]))
    return _mlp(feat, w1p.astype(jnp.bfloat16), b1, w2, b2)


# speedup vs baseline: 15.8330x; 1.5933x over previous
import functools

import jax
import jax.numpy as jnp
from jax.experimental import pallas as pl
from jax.experimental.pallas import tpu as pltpu


def _l0_kernel(z_ref, w_ref, b_ref, o_ref, col_ref):
    s = o_ref.shape[1]
    c4 = z_ref.shape[3]
    c = o_ref.shape[3]
    m = s * s
    for t, (p, q) in enumerate(((0, 0), (0, 1), (1, 0), (1, 1))):
        col_ref[:, t * c4:(t + 1) * c4] = (
            z_ref[0, p:p + s, q:q + s, :].reshape(m, c4))
    acc = jnp.dot(col_ref[...], w_ref[...], preferred_element_type=jnp.float32)
    mx = jnp.maximum(jnp.maximum(acc[:, :c], acc[:, c:2 * c]),
                     jnp.maximum(acc[:, 2 * c:3 * c], acc[:, 3 * c:]))
    mx = jnp.maximum(mx + b_ref[...], 0.0)
    o_ref[0] = mx.reshape(s, s, c).astype(o_ref.dtype)
    co = c // 4
    zr = jnp.zeros((1, s, 2 * co), o_ref.dtype)
    zc = jnp.zeros((s, 1, co), o_ref.dtype)
    o_ref[0, 0:1, :, 0:2 * co] = zr
    o_ref[0, s - 1:s, :, 2 * co:] = zr
    o_ref[0, :, 0:1, 0:co] = zc
    o_ref[0, :, 0:1, 2 * co:3 * co] = zc
    o_ref[0, :, s - 1:s, co:2 * co] = zc
    o_ref[0, :, s - 1:s, 3 * co:] = zc


def _l0_conv(z, w4, bias):
    B, s1 = z.shape[0], z.shape[1]
    s = s1 - 1
    c = w4.shape[1] // 4
    return pl.pallas_call(
        _l0_kernel,
        out_shape=jax.ShapeDtypeStruct((B, s, s, c), jnp.bfloat16),
        grid=(B,),
        in_specs=[
            pl.BlockSpec((1, s1, s1, z.shape[3]), lambda b: (b, 0, 0, 0)),
            pl.BlockSpec(w4.shape, lambda b: (0, 0)),
            pl.BlockSpec((1, c), lambda b: (0, 0)),
        ],
        out_specs=pl.BlockSpec((1, s, s, c), lambda b: (b, 0, 0, 0)),
        scratch_shapes=[pltpu.VMEM((s * s, w4.shape[0]), jnp.bfloat16)],
        compiler_params=pltpu.CompilerParams(
            dimension_semantics=("parallel",),
        ),
    )(z, w4, jnp.tile(bias, 4).reshape(1, c).astype(jnp.float32))


def _fold_weights_l0(w):
    cin, cout = w.shape[2], w.shape[3]
    zero = jnp.zeros((cin, cout), w.dtype)
    taps = []
    for p in (0, 1):
        for q in (0, 1):
            rows = []
            for aa in range(4):
                for bb in range(4):
                    cols = []
                    for dh in (0, 1):
                        for dw in (0, 1):
                            for al in (0, 1):
                                for be in (0, 1):
                                    kh = 4 * p + aa - 2 * al - dh - 2
                                    kw = 4 * q + bb - 2 * be - dw - 2
                                    ok = 0 <= kh <= 2 and 0 <= kw <= 2
                                    cols.append(w[kh, kw] if ok else zero)
                    rows.append(jnp.concatenate(cols, axis=1))
            taps.append(jnp.concatenate(rows, axis=0))
    return jnp.concatenate(taps, axis=0).astype(jnp.bfloat16)


def _s2d_conv_pool_kernel(z_ref, w_ref, b_ref, o_ref, col_ref):
    hp, wp, cout = o_ref.shape[1], o_ref.shape[2], o_ref.shape[3]
    c4 = z_ref.shape[3]
    mp = hp * wp

    for t, (p, q) in enumerate(((0, 0), (0, 1), (1, 0), (1, 1))):
        col_ref[:, t * c4:(t + 1) * c4] = (
            z_ref[0, p:p + hp, q:q + wp, :].reshape(mp, c4))

    acc = jnp.dot(col_ref[...], w_ref[...], preferred_element_type=jnp.float32)

    m = jnp.maximum(jnp.maximum(acc[:, :cout], acc[:, cout:2 * cout]),
                    jnp.maximum(acc[:, 2 * cout:3 * cout], acc[:, 3 * cout:]))
    m = jnp.maximum(m + b_ref[...], 0.0)
    o_ref[0] = m.reshape(hp, wp, cout).astype(o_ref.dtype)


def _s2d_conv_pool(z, w2, bias, hp, wp, cout, vmem_mb):
    B = z.shape[0]
    k16 = w2.shape[0]
    return pl.pallas_call(
        _s2d_conv_pool_kernel,
        out_shape=jax.ShapeDtypeStruct((B, hp, wp, cout), jnp.bfloat16),
        grid=(B,),
        in_specs=[
            pl.BlockSpec((1, z.shape[1], z.shape[2], z.shape[3]),
                         lambda b: (b, 0, 0, 0)),
            pl.BlockSpec((k16, 4 * cout), lambda b: (0, 0)),
            pl.BlockSpec((1, cout), lambda b: (0, 0)),
        ],
        out_specs=pl.BlockSpec((1, hp, wp, cout), lambda b: (b, 0, 0, 0)),
        scratch_shapes=[pltpu.VMEM((hp * wp, k16), jnp.bfloat16)],
        compiler_params=pltpu.CompilerParams(
            dimension_semantics=("parallel",),
            vmem_limit_bytes=vmem_mb * 1024 * 1024,
        ),
    )(z, w2, bias.reshape(1, cout).astype(jnp.float32))


def _fold_weights(w):
    cin, cout = w.shape[2], w.shape[3]
    zero = jnp.zeros((cin, cout), w.dtype)
    taps = []
    for p in (0, 1):
        for q in (0, 1):
            rows = []
            for a in (0, 1):
                for b in (0, 1):
                    cols = []
                    for dh in (0, 1):
                        for dw in (0, 1):
                            kh = 2 * p + a - dh
                            kw = 2 * q + b - dw
                            ok = 0 <= kh <= 2 and 0 <= kw <= 2
                            cols.append(w[kh, kw] if ok else zero)
                    rows.append(jnp.concatenate(cols, axis=1))
            taps.append(jnp.concatenate(rows, axis=0))
    return jnp.concatenate(taps, axis=0).astype(jnp.bfloat16)


def _s2d_nhwc(y):
    B, H, W, C = y.shape
    yp = jnp.pad(y, ((0, 0), (1, 1), (1, 1), (0, 0)))
    z = yp.reshape(B, (H + 2) // 2, 2, (W + 2) // 2, 2, C)
    return jnp.transpose(z, (0, 1, 3, 2, 4, 5)).reshape(
        B, (H + 2) // 2, (W + 2) // 2, 4 * C)


def _mlp_kernel(x_ref, w1_ref, b1_ref, w2_ref, b2_ref, o_ref):
    h = jnp.dot(x_ref[...], w1_ref[...], preferred_element_type=jnp.float32)
    h = jnp.maximum(h + b1_ref[...], 0.0)
    o_ref[...] = jnp.dot(h, w2_ref[...],
                         preferred_element_type=jnp.float32) + b2_ref[...]


def _mlp(x, w1p, b1, w2, b2):
    B, K = x.shape
    n1, n2 = w1p.shape[1], w2.shape[1]
    return pl.pallas_call(
        _mlp_kernel,
        out_shape=jax.ShapeDtypeStruct((B, n2), jnp.float32),
        grid=(1,),
        in_specs=[
            pl.BlockSpec((B, K), lambda i: (0, 0)),
            pl.BlockSpec((K, n1), lambda i: (0, 0)),
            pl.BlockSpec((1, n1), lambda i: (0, 0)),
            pl.BlockSpec((n1, n2), lambda i: (0, 0)),
            pl.BlockSpec((1, n2), lambda i: (0, 0)),
        ],
        out_specs=pl.BlockSpec((B, n2), lambda i: (0, 0)),
        compiler_params=pltpu.CompilerParams(
            dimension_semantics=("arbitrary",),
        ),
    )(x, w1p, b1.reshape(1, n1).astype(jnp.float32),
      w2.astype(jnp.float32), b2.reshape(1, n2).astype(jnp.float32))


@functools.partial(jax.jit, static_argnums=())
def kernel(x, cw0, cb0, cw1, cb1, cw2, cb2, cw3, cb3, cw4, cb4, w1, b1, w2, b2):
    B, _, H, W = x.shape

    s1 = (H + 8) // 4
    xb = x.astype(jnp.bfloat16)
    xp = jnp.pad(xb, ((0, 0), (0, 0), (5, 3), (5, 3)))
    z = xp.reshape(B, 3, s1, 4, s1, 4)
    z = jnp.transpose(z, (0, 2, 4, 3, 5, 1))
    z = z.reshape(B, s1, s1, 48)

    z = _l0_conv(z, _fold_weights_l0(cw0), cb0)

    convs = [(cw1, cb1), (cw2, cb2), (cw3, cb3), (cw4, cb4)]
    hp, wp = H // 4, W // 4
    for i, (cw, cb) in enumerate(convs):
        w2d = _fold_weights(cw)
        y = _s2d_conv_pool(z, w2d, cb, hp, wp, cw.shape[3], 32)
        if i < len(convs) - 1:
            z = _s2d_nhwc(y)
            hp, wp = hp // 2, wp // 2

    feat = y.reshape(B, -1)
    cout = cw4.shape[3]
    hw = y.shape[1] * y.shape[2]
    w1p = (w1.reshape(cout, y.shape[1], y.shape[2], w1.shape[1])
           .transpose(1, 2, 0, 3).reshape(hw * cout, w1.shape[1]))
    return _mlp(feat, w1p.astype(jnp.bfloat16), b1, w2, b2)


# speedup vs baseline: 19.0320x; 1.2020x over previous
import functools

import jax
import jax.numpy as jnp
from jax.experimental import pallas as pl
from jax.experimental.pallas import tpu as pltpu


def _l0_kernel(z_ref, w_ref, b_ref, o_ref, col_ref):
    s = o_ref.shape[1]
    c4 = z_ref.shape[3]
    c = o_ref.shape[3]
    wm = col_ref.shape[0] // s
    for t, (p, q) in enumerate(((0, 0), (0, 1), (1, 0), (1, 1))):
        col_ref[:, t * c4:(t + 1) * c4] = (
            z_ref[0, p:p + s, q:q + wm, :].reshape(s * wm, c4))
    acc = jnp.dot(col_ref[...], w_ref[...], preferred_element_type=jnp.float32)
    mx = jnp.maximum(jnp.maximum(acc[:, :c], acc[:, c:2 * c]),
                     jnp.maximum(acc[:, 2 * c:3 * c], acc[:, 3 * c:]))
    mx = jnp.maximum(mx + b_ref[...], 0.0)
    o_ref[0, :, 0:wm, :] = mx.reshape(s, wm, c).astype(o_ref.dtype)
    co = c // 4
    zr = jnp.zeros((1, s, 2 * co), o_ref.dtype)
    zc = jnp.zeros((s, 1, co), o_ref.dtype)
    o_ref[0, 0:1, 0:s, 0:2 * co] = zr
    o_ref[0, s - 1:s, 0:s, 2 * co:] = zr
    o_ref[0, :, 0:1, 0:co] = zc
    o_ref[0, :, 0:1, 2 * co:3 * co] = zc
    o_ref[0, :, s - 1:s, co:2 * co] = zc
    o_ref[0, :, s - 1:s, 3 * co:] = zc


def _l0_conv(z, w4, bias, s, wm, wo):
    B = z.shape[0]
    c = w4.shape[1] // 4
    return pl.pallas_call(
        _l0_kernel,
        out_shape=jax.ShapeDtypeStruct((B, s, wo, c), jnp.bfloat16),
        grid=(B,),
        in_specs=[
            pl.BlockSpec((1,) + z.shape[1:], lambda b: (b, 0, 0, 0)),
            pl.BlockSpec(w4.shape, lambda b: (0, 0)),
            pl.BlockSpec((1, c), lambda b: (0, 0)),
        ],
        out_specs=pl.BlockSpec((1, s, wo, c), lambda b: (b, 0, 0, 0)),
        scratch_shapes=[pltpu.VMEM((s * wm, w4.shape[0]), jnp.bfloat16)],
        compiler_params=pltpu.CompilerParams(
            dimension_semantics=("parallel",),
        ),
    )(z, w4, jnp.tile(bias, 4).reshape(1, c).astype(jnp.float32))


def _fold_weights_l0(w):
    cin, cout = w.shape[2], w.shape[3]
    zero = jnp.zeros((cin, cout), w.dtype)
    taps = []
    for p in (0, 1):
        for q in (0, 1):
            rows = []
            for aa in range(4):
                for bb in range(4):
                    cols = []
                    for dh in (0, 1):
                        for dw in (0, 1):
                            for al in (0, 1):
                                for be in (0, 1):
                                    kh = 4 * p + aa - 2 * al - dh - 2
                                    kw = 4 * q + bb - 2 * be - dw - 2
                                    ok = 0 <= kh <= 2 and 0 <= kw <= 2
                                    cols.append(w[kh, kw] if ok else zero)
                    rows.append(jnp.concatenate(cols, axis=1))
            taps.append(jnp.concatenate(rows, axis=0))
    return jnp.concatenate(taps, axis=0).astype(jnp.bfloat16)


def _s2d_conv_pool_kernel(z_ref, w_ref, b_ref, o_ref, col_ref, *, wp, tout):
    hp = o_ref.shape[2] // wp if tout else o_ref.shape[1]
    cout = b_ref.shape[1]
    c4 = z_ref.shape[3]
    wm = col_ref.shape[0] // hp

    for t, (p, q) in enumerate(((0, 0), (0, 1), (1, 0), (1, 1))):
        col_ref[:, t * c4:(t + 1) * c4] = (
            z_ref[0, p:p + hp, q:q + wm, :].reshape(hp * wm, c4))

    acc = jnp.dot(col_ref[...], w_ref[...], preferred_element_type=jnp.float32)

    m = jnp.maximum(jnp.maximum(acc[:, :cout], acc[:, cout:2 * cout]),
                    jnp.maximum(acc[:, 2 * cout:3 * cout], acc[:, 3 * cout:]))
    m = jnp.maximum(m + b_ref[...], 0.0).astype(o_ref.dtype)
    if tout:
        o_ref[0] = m.transpose(1, 0)
    else:
        o_ref[0, :, 0:wm, :] = m.reshape(hp, wm, cout)


def _s2d_conv_pool(z, w2, bias, hp, wp, cout, wm=None, wo=None, tout=False):
    B = z.shape[0]
    k16 = w2.shape[0]
    wm = wp if wm is None else wm
    wo = wm if wo is None else wo
    if tout:
        out_spec = pl.BlockSpec((1, cout, hp * wp), lambda b: (b, 0, 0))
        out_shape = jax.ShapeDtypeStruct((B, cout, hp * wp), jnp.bfloat16)
    else:
        out_spec = pl.BlockSpec((1, hp, wo, cout), lambda b: (b, 0, 0, 0))
        out_shape = jax.ShapeDtypeStruct((B, hp, wo, cout), jnp.bfloat16)
    body = functools.partial(_s2d_conv_pool_kernel, wp=wp, tout=tout)
    return pl.pallas_call(
        body,
        out_shape=out_shape,
        grid=(B,),
        in_specs=[
            pl.BlockSpec((1,) + z.shape[1:], lambda b: (b, 0, 0, 0)),
            pl.BlockSpec((k16, 4 * cout), lambda b: (0, 0)),
            pl.BlockSpec((1, cout), lambda b: (0, 0)),
        ],
        out_specs=out_spec,
        scratch_shapes=[pltpu.VMEM((hp * wm, k16), jnp.bfloat16)],
        compiler_params=pltpu.CompilerParams(
            dimension_semantics=("parallel",),
        ),
    )(z, w2, bias.reshape(1, cout).astype(jnp.float32))


def _fold_weights(w):
    cin, cout = w.shape[2], w.shape[3]
    zero = jnp.zeros((cin, cout), w.dtype)
    taps = []
    for p in (0, 1):
        for q in (0, 1):
            rows = []
            for a in (0, 1):
                for b in (0, 1):
                    cols = []
                    for dh in (0, 1):
                        for dw in (0, 1):
                            kh = 2 * p + a - dh
                            kw = 2 * q + b - dw
                            ok = 0 <= kh <= 2 and 0 <= kw <= 2
                            cols.append(w[kh, kw] if ok else zero)
                    rows.append(jnp.concatenate(cols, axis=1))
            taps.append(jnp.concatenate(rows, axis=0))
    return jnp.concatenate(taps, axis=0).astype(jnp.bfloat16)


def _s2d_nhwc(y):
    B, H, W, C = y.shape
    yp = jnp.pad(y, ((0, 0), (1, 1), (1, 1), (0, 0)))
    z = yp.reshape(B, (H + 2) // 2, 2, (W + 2) // 2, 2, C)
    return jnp.transpose(z, (0, 1, 3, 2, 4, 5)).reshape(
        B, (H + 2) // 2, (W + 2) // 2, 4 * C)


def _mlp_kernel(x_ref, w1_ref, b1_ref, w2_ref, b2_ref, o_ref):
    h = jnp.dot(x_ref[...], w1_ref[...], preferred_element_type=jnp.float32)
    h = jnp.maximum(h + b1_ref[...], 0.0)
    o_ref[...] = jnp.dot(h, w2_ref[...],
                         preferred_element_type=jnp.float32) + b2_ref[...]


def _mlp(x, w1p, b1, w2, b2):
    B, K = x.shape
    n1, n2 = w1p.shape[1], w2.shape[1]
    return pl.pallas_call(
        _mlp_kernel,
        out_shape=jax.ShapeDtypeStruct((B, n2), jnp.float32),
        grid=(1,),
        in_specs=[
            pl.BlockSpec((B, K), lambda i: (0, 0)),
            pl.BlockSpec((K, n1), lambda i: (0, 0)),
            pl.BlockSpec((1, n1), lambda i: (0, 0)),
            pl.BlockSpec((n1, n2), lambda i: (0, 0)),
            pl.BlockSpec((1, n2), lambda i: (0, 0)),
        ],
        out_specs=pl.BlockSpec((B, n2), lambda i: (0, 0)),
        compiler_params=pltpu.CompilerParams(
            dimension_semantics=("arbitrary",),
        ),
    )(x, w1p, b1.reshape(1, n1).astype(jnp.float32),
      w2.astype(jnp.float32), b2.reshape(1, n2).astype(jnp.float32))


@functools.partial(jax.jit, static_argnums=())
def kernel(x, cw0, cb0, cw1, cb1, cw2, cb2, cw3, cb3, cw4, cb4, w1, b1, w2, b2):
    B, _, H, W = x.shape

    s = H // 4 + 1
    wm0 = -(-s // 16) * 16
    xb = x.astype(jnp.bfloat16)
    xp = jnp.pad(xb, ((0, 0), (0, 0), (5, 4 * (s + 1) - H - 5),
                      (5, 4 * (wm0 + 1) - W - 5)))
    z = xp.reshape(B, 3, s + 1, 4, wm0 + 1, 4)
    z = jnp.transpose(z, (0, 2, 4, 3, 5, 1))
    z = z.reshape(B, s + 1, wm0 + 1, 48)

    z = _l0_conv(z, _fold_weights_l0(cw0), cb0, s, wm0, wm0 + 8)

    convs = [(cw1, cb1), (cw2, cb2), (cw3, cb3), (cw4, cb4)]
    hp = wp = H // 4
    for i, (cw, cb) in enumerate(convs):
        w2d = _fold_weights(cw)
        last = i == len(convs) - 1
        wm = -(-wp // 16) * 16 if i == 0 else wp
        y = _s2d_conv_pool(z, w2d, cb, hp, wp, cw.shape[3],
                           wm=wm, tout=last)
        if not last:
            z = _s2d_nhwc(y[:, :, :wp, :])
            hp, wp = hp // 2, wp // 2

    feat = y.reshape(B, -1)
    return _mlp(feat, w1.astype(jnp.bfloat16), b1, w2, b2)


# speedup vs baseline: 21.5050x; 1.1299x over previous
import functools

import jax
import jax.numpy as jnp
from jax.experimental import pallas as pl
from jax.experimental.pallas import tpu as pltpu


def _l0_kernel(z_ref, w_ref, b_ref, o_ref, col_ref):
    s = o_ref.shape[1]
    c4 = z_ref.shape[3]
    c = o_ref.shape[3]
    wm = col_ref.shape[0] // s
    for t, (p, q) in enumerate(((0, 0), (0, 1), (1, 0), (1, 1))):
        col_ref[:, t * c4:(t + 1) * c4] = (
            z_ref[0, p:p + s, q:q + wm, :].reshape(s * wm, c4))
    acc = jnp.dot(col_ref[...], w_ref[...], preferred_element_type=jnp.float32)
    mx = jnp.maximum(jnp.maximum(acc[:, :c], acc[:, c:2 * c]),
                     jnp.maximum(acc[:, 2 * c:3 * c], acc[:, 3 * c:]))
    mx = jnp.maximum(mx + b_ref[...], 0.0)
    o_ref[0, :, 0:wm, :] = mx.reshape(s, wm, c).astype(o_ref.dtype)
    co = c // 4
    zr = jnp.zeros((1, s, 2 * co), o_ref.dtype)
    zc = jnp.zeros((s, 1, co), o_ref.dtype)
    o_ref[0, 0:1, 0:s, 0:2 * co] = zr
    o_ref[0, s - 1:s, 0:s, 2 * co:] = zr
    o_ref[0, :, 0:1, 0:co] = zc
    o_ref[0, :, 0:1, 2 * co:3 * co] = zc
    o_ref[0, :, s - 1:s, co:2 * co] = zc
    o_ref[0, :, s - 1:s, 3 * co:] = zc


def _l0_conv(z, w4, bias, s, wm, wo):
    B = z.shape[0]
    c = w4.shape[1] // 4
    return pl.pallas_call(
        _l0_kernel,
        out_shape=jax.ShapeDtypeStruct((B, s, wo, c), jnp.bfloat16),
        grid=(B,),
        in_specs=[
            pl.BlockSpec((1,) + z.shape[1:], lambda b: (b, 0, 0, 0)),
            pl.BlockSpec(w4.shape, lambda b: (0, 0)),
            pl.BlockSpec((1, c), lambda b: (0, 0)),
        ],
        out_specs=pl.BlockSpec((1, s, wo, c), lambda b: (b, 0, 0, 0)),
        scratch_shapes=[pltpu.VMEM((s * wm, w4.shape[0]), jnp.bfloat16)],
        compiler_params=pltpu.CompilerParams(
            dimension_semantics=("parallel",),
        ),
    )(z, w4, jnp.tile(bias, 4).reshape(1, c).astype(jnp.float32))


def _fold_weights_l0(w):
    cin, cout = w.shape[2], w.shape[3]
    zero = jnp.zeros((cin, cout), w.dtype)
    taps = []
    for p in (0, 1):
        for q in (0, 1):
            rows = []
            for aa in range(4):
                for bb in range(4):
                    cols = []
                    for dh in (0, 1):
                        for dw in (0, 1):
                            for al in (0, 1):
                                for be in (0, 1):
                                    kh = 4 * p + aa - 2 * al - dh - 2
                                    kw = 4 * q + bb - 2 * be - dw - 2
                                    ok = 0 <= kh <= 2 and 0 <= kw <= 2
                                    cols.append(w[kh, kw] if ok else zero)
                    rows.append(jnp.concatenate(cols, axis=1))
            taps.append(jnp.concatenate(rows, axis=0))
    return jnp.concatenate(taps, axis=0).astype(jnp.bfloat16)


def _s2d_conv_pool_kernel(z_ref, w_ref, b_ref, o_ref, col_ref, *, wp, tout):
    hp = o_ref.shape[2] // wp if tout else o_ref.shape[1]
    cout = b_ref.shape[1]
    c4 = z_ref.shape[3]
    wm = col_ref.shape[0] // hp

    for t, (p, q) in enumerate(((0, 0), (0, 1), (1, 0), (1, 1))):
        col_ref[:, t * c4:(t + 1) * c4] = (
            z_ref[0, p:p + hp, q:q + wm, :].reshape(hp * wm, c4))

    acc = jnp.dot(col_ref[...], w_ref[...], preferred_element_type=jnp.float32)

    m = jnp.maximum(jnp.maximum(acc[:, :cout], acc[:, cout:2 * cout]),
                    jnp.maximum(acc[:, 2 * cout:3 * cout], acc[:, 3 * cout:]))
    m = jnp.maximum(m + b_ref[...], 0.0).astype(o_ref.dtype)
    if tout:
        o_ref[0] = m.transpose(1, 0)
    else:
        o_ref[0, :, 0:wm, :] = m.reshape(hp, wm, cout)


def _s2d_conv_pool(z, w2, bias, hp, wp, cout, wm=None, wo=None, tout=False):
    B = z.shape[0]
    k16 = w2.shape[0]
    wm = wp if wm is None else wm
    wo = wm if wo is None else wo
    if tout:
        out_spec = pl.BlockSpec((1, cout, hp * wp), lambda b: (b, 0, 0))
        out_shape = jax.ShapeDtypeStruct((B, cout, hp * wp), jnp.bfloat16)
    else:
        out_spec = pl.BlockSpec((1, hp, wo, cout), lambda b: (b, 0, 0, 0))
        out_shape = jax.ShapeDtypeStruct((B, hp, wo, cout), jnp.bfloat16)
    body = functools.partial(_s2d_conv_pool_kernel, wp=wp, tout=tout)
    return pl.pallas_call(
        body,
        out_shape=out_shape,
        grid=(B,),
        in_specs=[
            pl.BlockSpec((1,) + z.shape[1:], lambda b: (b, 0, 0, 0)),
            pl.BlockSpec((k16, 4 * cout), lambda b: (0, 0)),
            pl.BlockSpec((1, cout), lambda b: (0, 0)),
        ],
        out_specs=out_spec,
        scratch_shapes=[pltpu.VMEM((hp * wm, k16), jnp.bfloat16)],
        compiler_params=pltpu.CompilerParams(
            dimension_semantics=("parallel",),
        ),
    )(z, w2, bias.reshape(1, cout).astype(jnp.float32))


def _fold_weights(w):
    cin, cout = w.shape[2], w.shape[3]
    zero = jnp.zeros((cin, cout), w.dtype)
    taps = []
    for p in (0, 1):
        for q in (0, 1):
            rows = []
            for a in (0, 1):
                for b in (0, 1):
                    cols = []
                    for dh in (0, 1):
                        for dw in (0, 1):
                            kh = 2 * p + a - dh
                            kw = 2 * q + b - dw
                            ok = 0 <= kh <= 2 and 0 <= kw <= 2
                            cols.append(w[kh, kw] if ok else zero)
                    rows.append(jnp.concatenate(cols, axis=1))
            taps.append(jnp.concatenate(rows, axis=0))
    return jnp.concatenate(taps, axis=0).astype(jnp.bfloat16)


def _s2d_nhwc(y):
    B, H, W, C = y.shape
    yp = jnp.pad(y, ((0, 0), (1, 1), (1, 1), (0, 0)))
    z = yp.reshape(B, (H + 2) // 2, 2, (W + 2) // 2, 2, C)
    return jnp.transpose(z, (0, 1, 3, 2, 4, 5)).reshape(
        B, (H + 2) // 2, (W + 2) // 2, 4 * C)


def _pool_bias_relu(acc, b_ref):
    c = acc.shape[1] // 4
    m = jnp.maximum(jnp.maximum(acc[:, :c], acc[:, c:2 * c]),
                    jnp.maximum(acc[:, 2 * c:3 * c], acc[:, 3 * c:]))
    return jnp.maximum(m + b_ref[...], 0.0)


def _conv_step(z, w_ref, b_ref, col_ref, hp):
    c4 = z.shape[2]
    for t, (p, q) in enumerate(((0, 0), (0, 1), (1, 0), (1, 1))):
        col_ref[:, t * c4:(t + 1) * c4] = (
            z[p:p + hp, q:q + hp, :].reshape(hp * hp, c4))
    acc = jnp.dot(col_ref[...], w_ref[...], preferred_element_type=jnp.float32)
    return _pool_bias_relu(acc, b_ref)


def _s2d_halo_store(z_ref, y_ref):
    s1, c = z_ref.shape[0], y_ref.shape[2]
    s = s1 - 1
    z_ref[...] = jnp.zeros(z_ref.shape, z_ref.dtype)
    for a in (0, 1):
        for b in (0, 1):
            blk = (a * 2 + b) * c
            z_ref[1 - a:s1 - a, 1 - b:s1 - b, blk:blk + c] = (
                y_ref[pl.Slice(1 - a, s, 2), pl.Slice(1 - b, s, 2), :]
                .astype(z_ref.dtype))


def _tail_kernel(z_ref, w2_ref, b2_ref, w3_ref, b3_ref, w4_ref, b4_ref,
                 o_ref, col2, y2s, z3, col3, y3s, z4, col4):
    h2 = z_ref.shape[1] - 1
    h3, h4 = h2 // 2, h2 // 4
    y2s[...] = _conv_step(z_ref[0], w2_ref, b2_ref, col2, h2).reshape(
        h2, h2, w2_ref.shape[1] // 4)
    _s2d_halo_store(z3, y2s)
    y3s[...] = _conv_step(z3[...], w3_ref, b3_ref, col3, h3).reshape(
        h3, h3, w3_ref.shape[1] // 4)
    _s2d_halo_store(z4, y3s)
    m4 = _conv_step(z4[...], w4_ref, b4_ref, col4, h4)
    o_ref[0] = m4.transpose(1, 0).astype(o_ref.dtype)


def _tail(z2, w2d2, b2, w2d3, b3, w2d4, b4):
    B, s2 = z2.shape[0], z2.shape[1]
    h2 = s2 - 1
    h3, h4 = h2 // 2, h2 // 4
    c2, c3, c4o = w2d2.shape[1] // 4, w2d3.shape[1] // 4, w2d4.shape[1] // 4
    ws = lambda b: (0, 0)
    return pl.pallas_call(
        _tail_kernel,
        out_shape=jax.ShapeDtypeStruct((B, c4o, h4 * h4), jnp.bfloat16),
        grid=(B,),
        in_specs=[
            pl.BlockSpec((1,) + z2.shape[1:], lambda b: (b, 0, 0, 0)),
            pl.BlockSpec(w2d2.shape, ws), pl.BlockSpec((1, c2), ws),
            pl.BlockSpec(w2d3.shape, ws), pl.BlockSpec((1, c3), ws),
            pl.BlockSpec(w2d4.shape, ws), pl.BlockSpec((1, c4o), ws),
        ],
        out_specs=pl.BlockSpec((1, c4o, h4 * h4), lambda b: (b, 0, 0)),
        scratch_shapes=[
            pltpu.VMEM((h2 * h2, w2d2.shape[0]), jnp.bfloat16),
            pltpu.VMEM((h2, h2, c2), jnp.float32),
            pltpu.VMEM((h3 + 1, h3 + 1, 4 * c2), jnp.bfloat16),
            pltpu.VMEM((h3 * h3, w2d3.shape[0]), jnp.bfloat16),
            pltpu.VMEM((h3, h3, c3), jnp.float32),
            pltpu.VMEM((h4 + 1, h4 + 1, 4 * c3), jnp.bfloat16),
            pltpu.VMEM((h4 * h4, w2d4.shape[0]), jnp.bfloat16),
        ],
        compiler_params=pltpu.CompilerParams(
            dimension_semantics=("parallel",),
        ),
    )(z2, w2d2, b2.reshape(1, c2).astype(jnp.float32),
      w2d3, b3.reshape(1, c3).astype(jnp.float32),
      w2d4, b4.reshape(1, c4o).astype(jnp.float32))


def _mlp_kernel(x_ref, w1_ref, b1_ref, w2_ref, b2_ref, o_ref):
    h = jnp.dot(x_ref[...], w1_ref[...], preferred_element_type=jnp.float32)
    h = jnp.maximum(h + b1_ref[...], 0.0)
    o_ref[...] = jnp.dot(h, w2_ref[...],
                         preferred_element_type=jnp.float32) + b2_ref[...]


def _mlp(x, w1p, b1, w2, b2):
    B, K = x.shape
    n1, n2 = w1p.shape[1], w2.shape[1]
    return pl.pallas_call(
        _mlp_kernel,
        out_shape=jax.ShapeDtypeStruct((B, n2), jnp.float32),
        grid=(1,),
        in_specs=[
            pl.BlockSpec((B, K), lambda i: (0, 0)),
            pl.BlockSpec((K, n1), lambda i: (0, 0)),
            pl.BlockSpec((1, n1), lambda i: (0, 0)),
            pl.BlockSpec((n1, n2), lambda i: (0, 0)),
            pl.BlockSpec((1, n2), lambda i: (0, 0)),
        ],
        out_specs=pl.BlockSpec((B, n2), lambda i: (0, 0)),
        compiler_params=pltpu.CompilerParams(
            dimension_semantics=("arbitrary",),
        ),
    )(x, w1p, b1.reshape(1, n1).astype(jnp.float32),
      w2.astype(jnp.float32), b2.reshape(1, n2).astype(jnp.float32))


@functools.partial(jax.jit, static_argnums=())
def kernel(x, cw0, cb0, cw1, cb1, cw2, cb2, cw3, cb3, cw4, cb4, w1, b1, w2, b2):
    B, _, H, W = x.shape

    s = H // 4 + 1
    wm0 = -(-s // 16) * 16
    xb = x.astype(jnp.bfloat16)
    xp = jnp.pad(xb, ((0, 0), (0, 0), (5, 4 * (s + 1) - H - 5),
                      (5, 4 * (wm0 + 1) - W - 5)))
    z = xp.reshape(B, 3, s + 1, 4, wm0 + 1, 4)
    z = jnp.transpose(z, (0, 2, 4, 3, 5, 1))
    z = z.reshape(B, s + 1, wm0 + 1, 48)

    z = _l0_conv(z, _fold_weights_l0(cw0), cb0, s, wm0, wm0 + 8)

    hp = wp = H // 4
    y1 = _s2d_conv_pool(z, _fold_weights(cw1), cb1, hp, wp, cw1.shape[3],
                        wm=-(-wp // 16) * 16)
    z2 = _s2d_nhwc(y1[:, :, :wp, :])
    y = _tail(z2, _fold_weights(cw2), cb2, _fold_weights(cw3), cb3,
              _fold_weights(cw4), cb4)

    feat = y.reshape(B, -1)
    return _mlp(feat, w1.astype(jnp.bfloat16), b1, w2, b2)


# speedup vs baseline: 25.0102x; 1.1630x over previous
import functools

import jax
import jax.numpy as jnp
from jax.experimental import pallas as pl
from jax.experimental.pallas import tpu as pltpu


def _l0_kernel(z_ref, w_ref, b_ref, o_ref, col_ref):
    s = o_ref.shape[1]
    c4 = z_ref.shape[3]
    c = o_ref.shape[3]
    wm = col_ref.shape[0] // s
    for t, (p, q) in enumerate(((0, 0), (0, 1), (1, 0), (1, 1))):
        col_ref[:, t * c4:(t + 1) * c4] = (
            z_ref[0, p:p + s, q:q + wm, :].reshape(s * wm, c4))
    acc = jnp.dot(col_ref[...], w_ref[...], preferred_element_type=jnp.float32)
    mx = jnp.maximum(jnp.maximum(acc[:, :c], acc[:, c:2 * c]),
                     jnp.maximum(acc[:, 2 * c:3 * c], acc[:, 3 * c:]))
    mx = jnp.maximum(mx + b_ref[...], 0.0)
    o_ref[0, :, 0:wm, :] = mx.reshape(s, wm, c).astype(o_ref.dtype)
    co = c // 4
    zr = jnp.zeros((1, s, 2 * co), o_ref.dtype)
    zc = jnp.zeros((s, 1, co), o_ref.dtype)
    o_ref[0, 0:1, 0:s, 0:2 * co] = zr
    o_ref[0, s - 1:s, 0:s, 2 * co:] = zr
    o_ref[0, :, 0:1, 0:co] = zc
    o_ref[0, :, 0:1, 2 * co:3 * co] = zc
    o_ref[0, :, s - 1:s, co:2 * co] = zc
    o_ref[0, :, s - 1:s, 3 * co:] = zc


def _l0_conv(z, w4, bias, s, wm, wo):
    B = z.shape[0]
    c = w4.shape[1] // 4
    return pl.pallas_call(
        _l0_kernel,
        out_shape=jax.ShapeDtypeStruct((B, s, wo, c), jnp.bfloat16),
        grid=(B,),
        in_specs=[
            pl.BlockSpec((1,) + z.shape[1:], lambda b: (b, 0, 0, 0)),
            pl.BlockSpec(w4.shape, lambda b: (0, 0)),
            pl.BlockSpec((1, c), lambda b: (0, 0)),
        ],
        out_specs=pl.BlockSpec((1, s, wo, c), lambda b: (b, 0, 0, 0)),
        scratch_shapes=[pltpu.VMEM((s * wm, w4.shape[0]), jnp.bfloat16)],
        compiler_params=pltpu.CompilerParams(
            dimension_semantics=("parallel",),
        ),
    )(z, w4, jnp.tile(bias, 4).reshape(1, c).astype(jnp.float32))


def _fold_weights_l0(w):
    cin, cout = w.shape[2], w.shape[3]
    zero = jnp.zeros((cin, cout), w.dtype)
    taps = []
    for p in (0, 1):
        for q in (0, 1):
            rows = []
            for aa in range(4):
                for bb in range(4):
                    cols = []
                    for dh in (0, 1):
                        for dw in (0, 1):
                            for al in (0, 1):
                                for be in (0, 1):
                                    kh = 4 * p + aa - 2 * al - dh - 2
                                    kw = 4 * q + bb - 2 * be - dw - 2
                                    ok = 0 <= kh <= 2 and 0 <= kw <= 2
                                    cols.append(w[kh, kw] if ok else zero)
                    rows.append(jnp.concatenate(cols, axis=1))
            taps.append(jnp.concatenate(rows, axis=0))
    return jnp.concatenate(taps, axis=0).astype(jnp.bfloat16)


def _s2d_conv_pool_kernel(z_ref, w_ref, b_ref, o_ref, col_ref, *, wp, tout):
    hp = o_ref.shape[2] // wp if tout else o_ref.shape[1]
    cout = b_ref.shape[1]
    c4 = z_ref.shape[3]
    wm = col_ref.shape[0] // hp

    for t, (p, q) in enumerate(((0, 0), (0, 1), (1, 0), (1, 1))):
        col_ref[:, t * c4:(t + 1) * c4] = (
            z_ref[0, p:p + hp, q:q + wm, :].reshape(hp * wm, c4))

    acc = jnp.dot(col_ref[...], w_ref[...], preferred_element_type=jnp.float32)

    m = jnp.maximum(jnp.maximum(acc[:, :cout], acc[:, cout:2 * cout]),
                    jnp.maximum(acc[:, 2 * cout:3 * cout], acc[:, 3 * cout:]))
    m = jnp.maximum(m + b_ref[...], 0.0).astype(o_ref.dtype)
    if tout:
        o_ref[0] = m.transpose(1, 0)
    else:
        o_ref[0, :, 0:wm, :] = m.reshape(hp, wm, cout)


def _s2d_conv_pool(z, w2, bias, hp, wp, cout, wm=None, wo=None, tout=False):
    B = z.shape[0]
    k16 = w2.shape[0]
    wm = wp if wm is None else wm
    wo = wm if wo is None else wo
    if tout:
        out_spec = pl.BlockSpec((1, cout, hp * wp), lambda b: (b, 0, 0))
        out_shape = jax.ShapeDtypeStruct((B, cout, hp * wp), jnp.bfloat16)
    else:
        out_spec = pl.BlockSpec((1, hp, wo, cout), lambda b: (b, 0, 0, 0))
        out_shape = jax.ShapeDtypeStruct((B, hp, wo, cout), jnp.bfloat16)
    body = functools.partial(_s2d_conv_pool_kernel, wp=wp, tout=tout)
    return pl.pallas_call(
        body,
        out_shape=out_shape,
        grid=(B,),
        in_specs=[
            pl.BlockSpec((1,) + z.shape[1:], lambda b: (b, 0, 0, 0)),
            pl.BlockSpec((k16, 4 * cout), lambda b: (0, 0)),
            pl.BlockSpec((1, cout), lambda b: (0, 0)),
        ],
        out_specs=out_spec,
        scratch_shapes=[pltpu.VMEM((hp * wm, k16), jnp.bfloat16)],
        compiler_params=pltpu.CompilerParams(
            dimension_semantics=("parallel",),
        ),
    )(z, w2, bias.reshape(1, cout).astype(jnp.float32))


def _fold_weights(w):
    cin, cout = w.shape[2], w.shape[3]
    zero = jnp.zeros((cin, cout), w.dtype)
    taps = []
    for p in (0, 1):
        for q in (0, 1):
            rows = []
            for a in (0, 1):
                for b in (0, 1):
                    cols = []
                    for dh in (0, 1):
                        for dw in (0, 1):
                            kh = 2 * p + a - dh
                            kw = 2 * q + b - dw
                            ok = 0 <= kh <= 2 and 0 <= kw <= 2
                            cols.append(w[kh, kw] if ok else zero)
                    rows.append(jnp.concatenate(cols, axis=1))
            taps.append(jnp.concatenate(rows, axis=0))
    return jnp.concatenate(taps, axis=0).astype(jnp.bfloat16)


def _s2d_nhwc(y):
    B, H, W, C = y.shape
    yp = jnp.pad(y, ((0, 0), (1, 1), (1, 1), (0, 0)))
    z = yp.reshape(B, (H + 2) // 2, 2, (W + 2) // 2, 2, C)
    return jnp.transpose(z, (0, 1, 3, 2, 4, 5)).reshape(
        B, (H + 2) // 2, (W + 2) // 2, 4 * C)


def _pool_bias_relu(acc, b_ref):
    c = acc.shape[1] // 4
    m = jnp.maximum(jnp.maximum(acc[:, :c], acc[:, c:2 * c]),
                    jnp.maximum(acc[:, 2 * c:3 * c], acc[:, 3 * c:]))
    return jnp.maximum(m + b_ref[...], 0.0)


def _conv_step(z, w_ref, b_ref, col_ref, hp):
    c4 = z.shape[2]
    for t, (p, q) in enumerate(((0, 0), (0, 1), (1, 0), (1, 1))):
        col_ref[:, t * c4:(t + 1) * c4] = (
            z[p:p + hp, q:q + hp, :].reshape(hp * hp, c4))
    acc = jnp.dot(col_ref[...], w_ref[...], preferred_element_type=jnp.float32)
    return _pool_bias_relu(acc, b_ref)


def _s2d_halo_store(z_ref, y_ref):
    s1, c = z_ref.shape[0], y_ref.shape[2]
    s = s1 - 1
    z_ref[...] = jnp.zeros(z_ref.shape, z_ref.dtype)
    for a in (0, 1):
        for b in (0, 1):
            blk = (a * 2 + b) * c
            z_ref[1 - a:s1 - a, 1 - b:s1 - b, blk:blk + c] = (
                y_ref[pl.Slice(1 - a, s, 2), pl.Slice(1 - b, s, 2), :]
                .astype(z_ref.dtype))


def _tail_kernel(z_ref, w1_ref, b1_ref, w2_ref, b2_ref, w3_ref, b3_ref,
                 w4_ref, b4_ref, o_ref, col1, y1s, z2, col2, y2s, z3, col3,
                 y3s, z4, col4):
    h1 = y1s.shape[0]
    h2 = h1 // 2
    h3, h4 = h2 // 2, h2 // 4
    wm = col1.shape[0] // h1
    c4 = z_ref.shape[3]
    for t, (p, q) in enumerate(((0, 0), (0, 1), (1, 0), (1, 1))):
        col1[:, t * c4:(t + 1) * c4] = (
            z_ref[0, p:p + h1, q:q + wm, :].reshape(h1 * wm, c4))
    acc = jnp.dot(col1[...], w1_ref[...], preferred_element_type=jnp.float32)
    c1 = w1_ref.shape[1] // 4
    y1s[...] = _pool_bias_relu(acc, b1_ref).reshape(h1, wm, c1)[:, :h1, :]
    _s2d_halo_store(z2, y1s)
    y2s[...] = _conv_step(z2[...], w2_ref, b2_ref, col2, h2).reshape(
        h2, h2, w2_ref.shape[1] // 4)
    _s2d_halo_store(z3, y2s)
    y3s[...] = _conv_step(z3[...], w3_ref, b3_ref, col3, h3).reshape(
        h3, h3, w3_ref.shape[1] // 4)
    _s2d_halo_store(z4, y3s)
    m4 = _conv_step(z4[...], w4_ref, b4_ref, col4, h4)
    o_ref[0] = m4.transpose(1, 0).astype(o_ref.dtype)


def _tail(z1, w2d1, b1c, w2d2, b2c, w2d3, b3c, w2d4, b4c):
    B, s1 = z1.shape[0], z1.shape[1]
    h1 = s1 - 1
    wm = -(-h1 // 16) * 16
    h2 = h1 // 2
    h3, h4 = h2 // 2, h2 // 4
    c1, c2 = w2d1.shape[1] // 4, w2d2.shape[1] // 4
    c3, c4o = w2d3.shape[1] // 4, w2d4.shape[1] // 4
    ws = lambda b: (0, 0)
    return pl.pallas_call(
        _tail_kernel,
        out_shape=jax.ShapeDtypeStruct((B, c4o, h4 * h4), jnp.bfloat16),
        grid=(B,),
        in_specs=[
            pl.BlockSpec((1,) + z1.shape[1:], lambda b: (b, 0, 0, 0)),
            pl.BlockSpec(w2d1.shape, ws), pl.BlockSpec((1, c1), ws),
            pl.BlockSpec(w2d2.shape, ws), pl.BlockSpec((1, c2), ws),
            pl.BlockSpec(w2d3.shape, ws), pl.BlockSpec((1, c3), ws),
            pl.BlockSpec(w2d4.shape, ws), pl.BlockSpec((1, c4o), ws),
        ],
        out_specs=pl.BlockSpec((1, c4o, h4 * h4), lambda b: (b, 0, 0)),
        scratch_shapes=[
            pltpu.VMEM((h1 * wm, w2d1.shape[0]), jnp.bfloat16),
            pltpu.VMEM((h1, h1, c1), jnp.float32),
            pltpu.VMEM((h2 + 1, h2 + 1, 4 * c1), jnp.bfloat16),
            pltpu.VMEM((h2 * h2, w2d2.shape[0]), jnp.bfloat16),
            pltpu.VMEM((h2, h2, c2), jnp.float32),
            pltpu.VMEM((h3 + 1, h3 + 1, 4 * c2), jnp.bfloat16),
            pltpu.VMEM((h3 * h3, w2d3.shape[0]), jnp.bfloat16),
            pltpu.VMEM((h3, h3, c3), jnp.float32),
            pltpu.VMEM((h4 + 1, h4 + 1, 4 * c3), jnp.bfloat16),
            pltpu.VMEM((h4 * h4, w2d4.shape[0]), jnp.bfloat16),
        ],
        compiler_params=pltpu.CompilerParams(
            dimension_semantics=("parallel",),
        ),
    )(z1, w2d1, b1c.reshape(1, c1).astype(jnp.float32),
      w2d2, b2c.reshape(1, c2).astype(jnp.float32),
      w2d3, b3c.reshape(1, c3).astype(jnp.float32),
      w2d4, b4c.reshape(1, c4o).astype(jnp.float32))


def _mlp_kernel(x_ref, w1_ref, b1_ref, w2_ref, b2_ref, o_ref):
    h = jnp.dot(x_ref[...], w1_ref[...], preferred_element_type=jnp.float32)
    h = jnp.maximum(h + b1_ref[...], 0.0)
    o_ref[...] = jnp.dot(h, w2_ref[...],
                         preferred_element_type=jnp.float32) + b2_ref[...]


def _mlp(x, w1p, b1, w2, b2):
    B, K = x.shape
    n1, n2 = w1p.shape[1], w2.shape[1]
    return pl.pallas_call(
        _mlp_kernel,
        out_shape=jax.ShapeDtypeStruct((B, n2), jnp.float32),
        grid=(1,),
        in_specs=[
            pl.BlockSpec((B, K), lambda i: (0, 0)),
            pl.BlockSpec((K, n1), lambda i: (0, 0)),
            pl.BlockSpec((1, n1), lambda i: (0, 0)),
            pl.BlockSpec((n1, n2), lambda i: (0, 0)),
            pl.BlockSpec((1, n2), lambda i: (0, 0)),
        ],
        out_specs=pl.BlockSpec((B, n2), lambda i: (0, 0)),
        compiler_params=pltpu.CompilerParams(
            dimension_semantics=("arbitrary",),
        ),
    )(x, w1p, b1.reshape(1, n1).astype(jnp.float32),
      w2.astype(jnp.float32), b2.reshape(1, n2).astype(jnp.float32))


@functools.partial(jax.jit, static_argnums=())
def kernel(x, cw0, cb0, cw1, cb1, cw2, cb2, cw3, cb3, cw4, cb4, w1, b1, w2, b2):
    B, _, H, W = x.shape

    s = H // 4 + 1
    wm0 = -(-s // 16) * 16
    xb = x.astype(jnp.bfloat16)
    xp = jnp.pad(xb, ((0, 0), (0, 0), (5, 4 * (s + 1) - H - 5),
                      (5, 4 * (wm0 + 1) - W - 5)))
    z = xp.reshape(B, 3, s + 1, 4, wm0 + 1, 4)
    z = jnp.transpose(z, (0, 2, 4, 3, 5, 1))
    z = z.reshape(B, s + 1, wm0 + 1, 48)

    z = _l0_conv(z, _fold_weights_l0(cw0), cb0, s, wm0, wm0 + 8)

    y = _tail(z, _fold_weights(cw1), cb1, _fold_weights(cw2), cb2,
              _fold_weights(cw3), cb3, _fold_weights(cw4), cb4)

    feat = y.reshape(B, -1)
    return _mlp(feat, w1.astype(jnp.bfloat16), b1, w2, b2)


# speedup vs baseline: 25.1349x; 1.0050x over previous
import functools

import jax
import jax.numpy as jnp
from jax.experimental import pallas as pl
from jax.experimental.pallas import tpu as pltpu


def _l0_kernel(z_ref, w_ref, b_ref, o_ref, col_ref):
    s = o_ref.shape[1]
    c4 = z_ref.shape[3]
    c = o_ref.shape[3]
    wm = col_ref.shape[0] // s
    for t, (p, q) in enumerate(((0, 0), (0, 1), (1, 0), (1, 1))):
        col_ref[:, t * c4:(t + 1) * c4] = (
            z_ref[0, p:p + s, q:q + wm, :].reshape(s * wm, c4))
    acc = jnp.dot(col_ref[...], w_ref[...], preferred_element_type=jnp.float32)
    mx = jnp.maximum(jnp.maximum(acc[:, :c], acc[:, c:2 * c]),
                     jnp.maximum(acc[:, 2 * c:3 * c], acc[:, 3 * c:]))
    mx = jnp.maximum(mx + b_ref[...], 0.0)
    o_ref[0, :, 0:wm, :] = mx.reshape(s, wm, c).astype(o_ref.dtype)
    co = c // 4
    zr = jnp.zeros((1, s, 2 * co), o_ref.dtype)
    zc = jnp.zeros((s, 1, co), o_ref.dtype)
    o_ref[0, 0:1, 0:s, 0:2 * co] = zr
    o_ref[0, s - 1:s, 0:s, 2 * co:] = zr
    o_ref[0, :, 0:1, 0:co] = zc
    o_ref[0, :, 0:1, 2 * co:3 * co] = zc
    o_ref[0, :, s - 1:s, co:2 * co] = zc
    o_ref[0, :, s - 1:s, 3 * co:] = zc


def _l0_conv(z, w4, bias, s, wm, wo):
    B = z.shape[0]
    c = w4.shape[1] // 4
    return pl.pallas_call(
        _l0_kernel,
        out_shape=jax.ShapeDtypeStruct((B, s, wo, c), jnp.bfloat16),
        grid=(B,),
        in_specs=[
            pl.BlockSpec((1,) + z.shape[1:], lambda b: (b, 0, 0, 0)),
            pl.BlockSpec(w4.shape, lambda b: (0, 0)),
            pl.BlockSpec((1, c), lambda b: (0, 0)),
        ],
        out_specs=pl.BlockSpec((1, s, wo, c), lambda b: (b, 0, 0, 0)),
        scratch_shapes=[pltpu.VMEM((s * wm, w4.shape[0]), jnp.bfloat16)],
        compiler_params=pltpu.CompilerParams(
            dimension_semantics=("parallel",),
        ),
    )(z, w4, jnp.tile(bias, 4).reshape(1, c).astype(jnp.float32))


def _fold_weights_l0(w):
    cin, cout = w.shape[2], w.shape[3]
    zero = jnp.zeros((cin, cout), w.dtype)
    taps = []
    for p in (0, 1):
        for q in (0, 1):
            rows = []
            for aa in range(4):
                for bb in range(4):
                    cols = []
                    for dh in (0, 1):
                        for dw in (0, 1):
                            for al in (0, 1):
                                for be in (0, 1):
                                    kh = 4 * p + aa - 2 * al - dh - 2
                                    kw = 4 * q + bb - 2 * be - dw - 2
                                    ok = 0 <= kh <= 2 and 0 <= kw <= 2
                                    cols.append(w[kh, kw] if ok else zero)
                    rows.append(jnp.concatenate(cols, axis=1))
            taps.append(jnp.concatenate(rows, axis=0))
    return jnp.concatenate(taps, axis=0).astype(jnp.bfloat16)


def _s2d_conv_pool_kernel(z_ref, w_ref, b_ref, o_ref, col_ref, *, wp, tout):
    hp = o_ref.shape[2] // wp if tout else o_ref.shape[1]
    cout = b_ref.shape[1]
    c4 = z_ref.shape[3]
    wm = col_ref.shape[0] // hp

    for t, (p, q) in enumerate(((0, 0), (0, 1), (1, 0), (1, 1))):
        col_ref[:, t * c4:(t + 1) * c4] = (
            z_ref[0, p:p + hp, q:q + wm, :].reshape(hp * wm, c4))

    acc = jnp.dot(col_ref[...], w_ref[...], preferred_element_type=jnp.float32)

    m = jnp.maximum(jnp.maximum(acc[:, :cout], acc[:, cout:2 * cout]),
                    jnp.maximum(acc[:, 2 * cout:3 * cout], acc[:, 3 * cout:]))
    m = jnp.maximum(m + b_ref[...], 0.0).astype(o_ref.dtype)
    if tout:
        o_ref[0] = m.transpose(1, 0)
    else:
        o_ref[0, :, 0:wm, :] = m.reshape(hp, wm, cout)


def _s2d_conv_pool(z, w2, bias, hp, wp, cout, wm=None, wo=None, tout=False):
    B = z.shape[0]
    k16 = w2.shape[0]
    wm = wp if wm is None else wm
    wo = wm if wo is None else wo
    if tout:
        out_spec = pl.BlockSpec((1, cout, hp * wp), lambda b: (b, 0, 0))
        out_shape = jax.ShapeDtypeStruct((B, cout, hp * wp), jnp.bfloat16)
    else:
        out_spec = pl.BlockSpec((1, hp, wo, cout), lambda b: (b, 0, 0, 0))
        out_shape = jax.ShapeDtypeStruct((B, hp, wo, cout), jnp.bfloat16)
    body = functools.partial(_s2d_conv_pool_kernel, wp=wp, tout=tout)
    return pl.pallas_call(
        body,
        out_shape=out_shape,
        grid=(B,),
        in_specs=[
            pl.BlockSpec((1,) + z.shape[1:], lambda b: (b, 0, 0, 0)),
            pl.BlockSpec((k16, 4 * cout), lambda b: (0, 0)),
            pl.BlockSpec((1, cout), lambda b: (0, 0)),
        ],
        out_specs=out_spec,
        scratch_shapes=[pltpu.VMEM((hp * wm, k16), jnp.bfloat16)],
        compiler_params=pltpu.CompilerParams(
            dimension_semantics=("parallel",),
        ),
    )(z, w2, bias.reshape(1, cout).astype(jnp.float32))


def _fold_weights(w):
    cin, cout = w.shape[2], w.shape[3]
    zero = jnp.zeros((cin, cout), w.dtype)
    taps = []
    for p in (0, 1):
        for q in (0, 1):
            rows = []
            for a in (0, 1):
                for b in (0, 1):
                    cols = []
                    for dh in (0, 1):
                        for dw in (0, 1):
                            kh = 2 * p + a - dh
                            kw = 2 * q + b - dw
                            ok = 0 <= kh <= 2 and 0 <= kw <= 2
                            cols.append(w[kh, kw] if ok else zero)
                    rows.append(jnp.concatenate(cols, axis=1))
            taps.append(jnp.concatenate(rows, axis=0))
    return jnp.concatenate(taps, axis=0).astype(jnp.bfloat16)


def _s2d_nhwc(y):
    B, H, W, C = y.shape
    yp = jnp.pad(y, ((0, 0), (1, 1), (1, 1), (0, 0)))
    z = yp.reshape(B, (H + 2) // 2, 2, (W + 2) // 2, 2, C)
    return jnp.transpose(z, (0, 1, 3, 2, 4, 5)).reshape(
        B, (H + 2) // 2, (W + 2) // 2, 4 * C)


def _pool_bias_relu(acc, b_ref):
    c = acc.shape[1] // 4
    m = jnp.maximum(jnp.maximum(acc[:, :c], acc[:, c:2 * c]),
                    jnp.maximum(acc[:, 2 * c:3 * c], acc[:, 3 * c:]))
    return jnp.maximum(m + b_ref[...], 0.0)


def _conv_step(z, w_ref, b_ref, col_ref, hp):
    c4 = z.shape[2]
    for t, (p, q) in enumerate(((0, 0), (0, 1), (1, 0), (1, 1))):
        col_ref[:, t * c4:(t + 1) * c4] = (
            z[p:p + hp, q:q + hp, :].reshape(hp * hp, c4))
    acc = jnp.dot(col_ref[...], w_ref[...], preferred_element_type=jnp.float32)
    return _pool_bias_relu(acc, b_ref)


def _s2d_halo_store(z_ref, y_ref):
    s1, c = z_ref.shape[0], y_ref.shape[2]
    s = s1 - 1
    z_ref[...] = jnp.zeros(z_ref.shape, z_ref.dtype)
    for a in (0, 1):
        for b in (0, 1):
            blk = (a * 2 + b) * c
            z_ref[1 - a:s1 - a, 1 - b:s1 - b, blk:blk + c] = (
                y_ref[pl.Slice(1 - a, s, 2), pl.Slice(1 - b, s, 2), :]
                .astype(z_ref.dtype))


def _tail_kernel(z_ref, w0_ref, b0_ref, w1_ref, b1_ref, w2_ref, b2_ref,
                 w3_ref, b3_ref, w4_ref, b4_ref, o_ref, col0, z1s, col1, y1s,
                 z2, col2, y2s, z3, col3, y3s, z4, col4):
    h1 = y1s.shape[0]
    h2 = h1 // 2
    h3, h4 = h2 // 2, h2 // 4
    c4in = z_ref.shape[3]
    s = z1s.shape[0]
    c0 = z1s.shape[2]
    wm0 = col0.shape[0] // s
    for t, (p, q) in enumerate(((0, 0), (0, 1), (1, 0), (1, 1))):
        col0[:, t * c4in:(t + 1) * c4in] = (
            z_ref[0, p:p + s, q:q + wm0, :].reshape(s * wm0, c4in))
    acc = jnp.dot(col0[...], w0_ref[...], preferred_element_type=jnp.float32)
    mx = _pool_bias_relu(acc, b0_ref).astype(z1s.dtype)
    z1s[:, 0:wm0, :] = mx.reshape(s, wm0, c0)
    co = c0 // 4
    zr = jnp.zeros((1, s, 2 * co), z1s.dtype)
    zc = jnp.zeros((s, 1, co), z1s.dtype)
    z1s[0:1, 0:s, 0:2 * co] = zr
    z1s[s - 1:s, 0:s, 2 * co:] = zr
    z1s[:, 0:1, 0:co] = zc
    z1s[:, 0:1, 2 * co:3 * co] = zc
    z1s[:, s - 1:s, co:2 * co] = zc
    z1s[:, s - 1:s, 3 * co:] = zc
    c4 = z1s.shape[2]
    wm = col1.shape[0] // h1
    for t, (p, q) in enumerate(((0, 0), (0, 1), (1, 0), (1, 1))):
        col1[:, t * c4:(t + 1) * c4] = (
            z1s[p:p + h1, q:q + wm, :].reshape(h1 * wm, c4))
    acc = jnp.dot(col1[...], w1_ref[...], preferred_element_type=jnp.float32)
    c1 = w1_ref.shape[1] // 4
    y1s[...] = _pool_bias_relu(acc, b1_ref).reshape(h1, wm, c1)[:, :h1, :]
    _s2d_halo_store(z2, y1s)
    y2s[...] = _conv_step(z2[...], w2_ref, b2_ref, col2, h2).reshape(
        h2, h2, w2_ref.shape[1] // 4)
    _s2d_halo_store(z3, y2s)
    y3s[...] = _conv_step(z3[...], w3_ref, b3_ref, col3, h3).reshape(
        h3, h3, w3_ref.shape[1] // 4)
    _s2d_halo_store(z4, y3s)
    m4 = _conv_step(z4[...], w4_ref, b4_ref, col4, h4)
    o_ref[0] = m4.transpose(1, 0).astype(o_ref.dtype)


def _tail(z, w2d0, b0c, w2d1, b1c, w2d2, b2c, w2d3, b3c, w2d4, b4c, s):
    B = z.shape[0]
    wm0 = -(-s // 16) * 16
    h1 = s - 1
    wm = -(-h1 // 16) * 16
    h2 = h1 // 2
    h3, h4 = h2 // 2, h2 // 4
    c0, c1 = w2d0.shape[1] // 4, w2d1.shape[1] // 4
    c2, c3 = w2d2.shape[1] // 4, w2d3.shape[1] // 4
    c4o = w2d4.shape[1] // 4
    ws = lambda b: (0, 0)
    return pl.pallas_call(
        _tail_kernel,
        out_shape=jax.ShapeDtypeStruct((B, c4o, h4 * h4), jnp.bfloat16),
        grid=(B,),
        in_specs=[
            pl.BlockSpec((1,) + z.shape[1:], lambda b: (b, 0, 0, 0)),
            pl.BlockSpec(w2d0.shape, ws), pl.BlockSpec((1, c0), ws),
            pl.BlockSpec(w2d1.shape, ws), pl.BlockSpec((1, c1), ws),
            pl.BlockSpec(w2d2.shape, ws), pl.BlockSpec((1, c2), ws),
            pl.BlockSpec(w2d3.shape, ws), pl.BlockSpec((1, c3), ws),
            pl.BlockSpec(w2d4.shape, ws), pl.BlockSpec((1, c4o), ws),
        ],
        out_specs=pl.BlockSpec((1, c4o, h4 * h4), lambda b: (b, 0, 0)),
        scratch_shapes=[
            pltpu.VMEM((s * wm0, w2d0.shape[0]), jnp.bfloat16),
            pltpu.VMEM((s, wm0 + 8, c0), jnp.bfloat16),
            pltpu.VMEM((h1 * wm, w2d1.shape[0]), jnp.bfloat16),
            pltpu.VMEM((h1, h1, c1), jnp.float32),
            pltpu.VMEM((h2 + 1, h2 + 1, 4 * c1), jnp.bfloat16),
            pltpu.VMEM((h2 * h2, w2d2.shape[0]), jnp.bfloat16),
            pltpu.VMEM((h2, h2, c2), jnp.float32),
            pltpu.VMEM((h3 + 1, h3 + 1, 4 * c2), jnp.bfloat16),
            pltpu.VMEM((h3 * h3, w2d3.shape[0]), jnp.bfloat16),
            pltpu.VMEM((h3, h3, c3), jnp.float32),
            pltpu.VMEM((h4 + 1, h4 + 1, 4 * c3), jnp.bfloat16),
            pltpu.VMEM((h4 * h4, w2d4.shape[0]), jnp.bfloat16),
        ],
        compiler_params=pltpu.CompilerParams(
            dimension_semantics=("parallel",),
            vmem_limit_bytes=48 * 1024 * 1024,
        ),
    )(z, w2d0, jnp.tile(b0c, 4).reshape(1, c0).astype(jnp.float32),
      w2d1, b1c.reshape(1, c1).astype(jnp.float32),
      w2d2, b2c.reshape(1, c2).astype(jnp.float32),
      w2d3, b3c.reshape(1, c3).astype(jnp.float32),
      w2d4, b4c.reshape(1, c4o).astype(jnp.float32))


def _mlp_kernel(x_ref, w1_ref, b1_ref, w2_ref, b2_ref, o_ref):
    x = x_ref[...].reshape(x_ref.shape[0], -1)
    h = jnp.dot(x, w1_ref[...], preferred_element_type=jnp.float32)
    h = jnp.maximum(h + b1_ref[...], 0.0)
    o_ref[...] = jnp.dot(h, w2_ref[...],
                         preferred_element_type=jnp.float32) + b2_ref[...]


def _mlp(x, w1p, b1, w2, b2):
    B, K = x.shape[0], x.shape[1] * x.shape[2]
    n1, n2 = w1p.shape[1], w2.shape[1]
    return pl.pallas_call(
        _mlp_kernel,
        out_shape=jax.ShapeDtypeStruct((B, n2), jnp.float32),
        grid=(1,),
        in_specs=[
            pl.BlockSpec(x.shape, lambda i: (0,) * len(x.shape)),
            pl.BlockSpec((K, n1), lambda i: (0, 0)),
            pl.BlockSpec((1, n1), lambda i: (0, 0)),
            pl.BlockSpec((n1, n2), lambda i: (0, 0)),
            pl.BlockSpec((1, n2), lambda i: (0, 0)),
        ],
        out_specs=pl.BlockSpec((B, n2), lambda i: (0, 0)),
        compiler_params=pltpu.CompilerParams(
            dimension_semantics=("arbitrary",),
        ),
    )(x, w1p, b1.reshape(1, n1).astype(jnp.float32),
      w2.astype(jnp.float32), b2.reshape(1, n2).astype(jnp.float32))


@functools.partial(jax.jit, static_argnums=())
def kernel(x, cw0, cb0, cw1, cb1, cw2, cb2, cw3, cb3, cw4, cb4, w1, b1, w2, b2):
    B, _, H, W = x.shape

    s = H // 4 + 1
    wm0 = -(-s // 16) * 16
    xb = x.astype(jnp.bfloat16)
    xp = jnp.pad(xb, ((0, 0), (0, 0), (5, 4 * (s + 1) - H - 5),
                      (5, 4 * (wm0 + 1) - W - 5)))
    z = xp.reshape(B, 3, s + 1, 4, wm0 + 1, 4)
    z = jnp.transpose(z, (0, 2, 4, 3, 5, 1))
    z = z.reshape(B, s + 1, wm0 + 1, 48)

    y = _tail(z, _fold_weights_l0(cw0), cb0, _fold_weights(cw1), cb1,
              _fold_weights(cw2), cb2, _fold_weights(cw3), cb3,
              _fold_weights(cw4), cb4, s)

    return _mlp(y, w1.astype(jnp.bfloat16), b1, w2, b2)


# speedup vs baseline: 25.1437x; 1.0003x over previous
import functools

import jax
import jax.numpy as jnp
from jax.experimental import pallas as pl
from jax.experimental.pallas import tpu as pltpu


def _fold_weights_l0(w):
    cin, cout = w.shape[2], w.shape[3]
    zero = jnp.zeros((cin, cout), w.dtype)
    taps = []
    for p in (0, 1):
        for q in (0, 1):
            rows = []
            for aa in range(4):
                for bb in range(4):
                    cols = []
                    for dh in (0, 1):
                        for dw in (0, 1):
                            for al in (0, 1):
                                for be in (0, 1):
                                    kh = 4 * p + aa - 2 * al - dh - 2
                                    kw = 4 * q + bb - 2 * be - dw - 2
                                    ok = 0 <= kh <= 2 and 0 <= kw <= 2
                                    cols.append(w[kh, kw] if ok else zero)
                    rows.append(jnp.concatenate(cols, axis=1))
            taps.append(jnp.concatenate(rows, axis=0))
    return jnp.concatenate(taps, axis=0).astype(jnp.bfloat16)


def _fold_weights(w):
    cin, cout = w.shape[2], w.shape[3]
    zero = jnp.zeros((cin, cout), w.dtype)
    taps = []
    for p in (0, 1):
        for q in (0, 1):
            rows = []
            for a in (0, 1):
                for b in (0, 1):
                    cols = []
                    for dh in (0, 1):
                        for dw in (0, 1):
                            kh = 2 * p + a - dh
                            kw = 2 * q + b - dw
                            ok = 0 <= kh <= 2 and 0 <= kw <= 2
                            cols.append(w[kh, kw] if ok else zero)
                    rows.append(jnp.concatenate(cols, axis=1))
            taps.append(jnp.concatenate(rows, axis=0))
    return jnp.concatenate(taps, axis=0).astype(jnp.bfloat16)


def _pool_bias_relu(acc, b_ref):
    c = acc.shape[1] // 4
    m = jnp.maximum(jnp.maximum(acc[:, :c], acc[:, c:2 * c]),
                    jnp.maximum(acc[:, 2 * c:3 * c], acc[:, 3 * c:]))
    return jnp.maximum(m + b_ref[...], 0.0)


def _conv_step(z, w_ref, b_ref, col_ref, hp):
    c4 = z.shape[2]
    for t, (p, q) in enumerate(((0, 0), (0, 1), (1, 0), (1, 1))):
        col_ref[:, t * c4:(t + 1) * c4] = (
            z[p:p + hp, q:q + hp, :].reshape(hp * hp, c4))
    acc = jnp.dot(col_ref[...], w_ref[...], preferred_element_type=jnp.float32)
    return _pool_bias_relu(acc, b_ref)


def _s2d_halo_store(z_ref, y_ref):
    s1, c = z_ref.shape[0], y_ref.shape[2]
    s = s1 - 1
    z_ref[...] = jnp.zeros(z_ref.shape, z_ref.dtype)
    for a in (0, 1):
        for b in (0, 1):
            blk = (a * 2 + b) * c
            z_ref[1 - a:s1 - a, 1 - b:s1 - b, blk:blk + c] = (
                y_ref[pl.Slice(1 - a, s, 2), pl.Slice(1 - b, s, 2), :]
                .astype(z_ref.dtype))


def _tail_kernel(z_ref, w0_ref, b0_ref, w1_ref, b1_ref, w2_ref, b2_ref,
                 w3_ref, b3_ref, w4_ref, b4_ref, o_ref, col0, z1s, col1, y1s,
                 z2, col2, y2s, z3, col3, y3s, z4, col4):
    h1 = y1s.shape[0]
    h2 = h1 // 2
    h3, h4 = h2 // 2, h2 // 4
    c4in = z_ref.shape[3]
    s = z1s.shape[0]
    c0 = z1s.shape[2]
    wm0 = col0.shape[0] // s
    for t, (p, q) in enumerate(((0, 0), (0, 1), (1, 0), (1, 1))):
        col0[:, t * c4in:(t + 1) * c4in] = (
            z_ref[0, p:p + s, q:q + wm0, :].reshape(s * wm0, c4in))
    acc = jnp.dot(col0[...], w0_ref[...], preferred_element_type=jnp.float32)
    mx = _pool_bias_relu(acc, b0_ref).astype(z1s.dtype)
    z1s[:, 0:wm0, :] = mx.reshape(s, wm0, c0)
    co = c0 // 4
    zr = jnp.zeros((1, s, 2 * co), z1s.dtype)
    zc = jnp.zeros((s, 1, co), z1s.dtype)
    z1s[0:1, 0:s, 0:2 * co] = zr
    z1s[s - 1:s, 0:s, 2 * co:] = zr
    z1s[:, 0:1, 0:co] = zc
    z1s[:, 0:1, 2 * co:3 * co] = zc
    z1s[:, s - 1:s, co:2 * co] = zc
    z1s[:, s - 1:s, 3 * co:] = zc
    c4 = z1s.shape[2]
    wm = col1.shape[0] // h1
    for t, (p, q) in enumerate(((0, 0), (0, 1), (1, 0), (1, 1))):
        col1[:, t * c4:(t + 1) * c4] = (
            z1s[p:p + h1, q:q + wm, :].reshape(h1 * wm, c4))
    acc = jnp.dot(col1[...], w1_ref[...], preferred_element_type=jnp.float32)
    c1 = w1_ref.shape[1] // 4
    y1s[...] = _pool_bias_relu(acc, b1_ref).reshape(h1, wm, c1)[:, :h1, :]
    _s2d_halo_store(z2, y1s)
    y2s[...] = _conv_step(z2[...], w2_ref, b2_ref, col2, h2).reshape(
        h2, h2, w2_ref.shape[1] // 4)
    _s2d_halo_store(z3, y2s)
    y3s[...] = _conv_step(z3[...], w3_ref, b3_ref, col3, h3).reshape(
        h3, h3, w3_ref.shape[1] // 4)
    _s2d_halo_store(z4, y3s)
    m4 = _conv_step(z4[...], w4_ref, b4_ref, col4, h4)
    o_ref[0] = m4.transpose(1, 0).astype(o_ref.dtype)


def _tail(z, w2d0, b0c, w2d1, b1c, w2d2, b2c, w2d3, b3c, w2d4, b4c, s):
    B = z.shape[0]
    wm0 = -(-s // 16) * 16
    h1 = s - 1
    wm = -(-h1 // 16) * 16
    h2 = h1 // 2
    h3, h4 = h2 // 2, h2 // 4
    c0, c1 = w2d0.shape[1] // 4, w2d1.shape[1] // 4
    c2, c3 = w2d2.shape[1] // 4, w2d3.shape[1] // 4
    c4o = w2d4.shape[1] // 4
    ws = lambda b: (0, 0)
    return pl.pallas_call(
        _tail_kernel,
        out_shape=jax.ShapeDtypeStruct((B, c4o, h4 * h4), jnp.bfloat16),
        grid=(B,),
        in_specs=[
            pl.BlockSpec((1,) + z.shape[1:], lambda b: (b, 0, 0, 0)),
            pl.BlockSpec(w2d0.shape, ws), pl.BlockSpec((1, c0), ws),
            pl.BlockSpec(w2d1.shape, ws), pl.BlockSpec((1, c1), ws),
            pl.BlockSpec(w2d2.shape, ws), pl.BlockSpec((1, c2), ws),
            pl.BlockSpec(w2d3.shape, ws), pl.BlockSpec((1, c3), ws),
            pl.BlockSpec(w2d4.shape, ws), pl.BlockSpec((1, c4o), ws),
        ],
        out_specs=pl.BlockSpec((1, c4o, h4 * h4), lambda b: (b, 0, 0)),
        scratch_shapes=[
            pltpu.VMEM((s * wm0, w2d0.shape[0]), jnp.bfloat16),
            pltpu.VMEM((s, wm0 + 8, c0), jnp.bfloat16),
            pltpu.VMEM((h1 * wm, w2d1.shape[0]), jnp.bfloat16),
            pltpu.VMEM((h1, h1, c1), jnp.float32),
            pltpu.VMEM((h2 + 1, h2 + 1, 4 * c1), jnp.bfloat16),
            pltpu.VMEM((h2 * h2, w2d2.shape[0]), jnp.bfloat16),
            pltpu.VMEM((h2, h2, c2), jnp.float32),
            pltpu.VMEM((h3 + 1, h3 + 1, 4 * c2), jnp.bfloat16),
            pltpu.VMEM((h3 * h3, w2d3.shape[0]), jnp.bfloat16),
            pltpu.VMEM((h3, h3, c3), jnp.float32),
            pltpu.VMEM((h4 + 1, h4 + 1, 4 * c3), jnp.bfloat16),
            pltpu.VMEM((h4 * h4, w2d4.shape[0]), jnp.bfloat16),
        ],
        compiler_params=pltpu.CompilerParams(
            dimension_semantics=("arbitrary",),
            vmem_limit_bytes=48 * 1024 * 1024,
        ),
    )(z, w2d0, jnp.tile(b0c, 4).reshape(1, c0).astype(jnp.float32),
      w2d1, b1c.reshape(1, c1).astype(jnp.float32),
      w2d2, b2c.reshape(1, c2).astype(jnp.float32),
      w2d3, b3c.reshape(1, c3).astype(jnp.float32),
      w2d4, b4c.reshape(1, c4o).astype(jnp.float32))


def _mlp_kernel(x_ref, w1_ref, b1_ref, w2_ref, b2_ref, o_ref):
    x = x_ref[...].reshape(x_ref.shape[0], -1)
    h = jnp.dot(x, w1_ref[...], preferred_element_type=jnp.float32)
    h = jnp.maximum(h + b1_ref[...], 0.0)
    o_ref[...] = jnp.dot(h, w2_ref[...],
                         preferred_element_type=jnp.float32) + b2_ref[...]


def _mlp(x, w1p, b1, w2, b2):
    B, K = x.shape[0], x.shape[1] * x.shape[2]
    n1, n2 = w1p.shape[1], w2.shape[1]
    return pl.pallas_call(
        _mlp_kernel,
        out_shape=jax.ShapeDtypeStruct((B, n2), jnp.float32),
        grid=(1,),
        in_specs=[
            pl.BlockSpec(x.shape, lambda i: (0,) * len(x.shape)),
            pl.BlockSpec((K, n1), lambda i: (0, 0)),
            pl.BlockSpec((1, n1), lambda i: (0, 0)),
            pl.BlockSpec((n1, n2), lambda i: (0, 0)),
            pl.BlockSpec((1, n2), lambda i: (0, 0)),
        ],
        out_specs=pl.BlockSpec((B, n2), lambda i: (0, 0)),
        compiler_params=pltpu.CompilerParams(
            dimension_semantics=("arbitrary",),
        ),
    )(x, w1p, b1.reshape(1, n1).astype(jnp.float32),
      w2.astype(jnp.float32), b2.reshape(1, n2).astype(jnp.float32))


@functools.partial(jax.jit, static_argnums=())
def kernel(x, cw0, cb0, cw1, cb1, cw2, cb2, cw3, cb3, cw4, cb4, w1, b1, w2, b2):
    B, _, H, W = x.shape

    s = H // 4 + 1
    wm0 = -(-s // 16) * 16
    xb = x.astype(jnp.bfloat16)
    xp = jnp.pad(xb, ((0, 0), (0, 0), (5, 4 * (s + 1) - H - 5),
                      (5, 4 * (wm0 + 1) - W - 5)))
    z = xp.reshape(B, 3, s + 1, 4, wm0 + 1, 4)
    z = jnp.transpose(z, (0, 2, 4, 3, 5, 1))
    z = z.reshape(B, s + 1, wm0 + 1, 48)

    y = _tail(z, _fold_weights_l0(cw0), cb0, _fold_weights(cw1), cb1,
              _fold_weights(cw2), cb2, _fold_weights(cw3), cb3,
              _fold_weights(cw4), cb4, s)

    return _mlp(y, w1.astype(jnp.bfloat16), b1, w2, b2)
